# baseline (speedup 1.0000x reference)
import functools

import jax
import jax.numpy as jnp
from jax import lax
from jax.experimental import pallas as pl
from jax.experimental.pallas import tpu as pltpu
from jax.experimental.pallas import tpu_sc as plsc

ACTIONS_MAX = 1000
EMB_SIZE = 128
NUM_AGENTS = 26
MLP_OUT = 1024
BATCH = 16384
K_DIM = NUM_AGENTS * EMB_SIZE

NB = 8
IPB = NB * NUM_AGENTS
HALF = IPB // 2

BM = 512
NUM_CHUNKS = 8
BC = BATCH // NUM_CHUNKS


def _sc_gather(idx, tab, bc):
    nblocks = bc // NB
    info = plsc.get_sparse_core_info()
    nc, ns = info.num_cores, info.num_subcores
    nw = nc * ns
    blocks_per_w = nblocks // nw
    assert nblocks % nw == 0 and blocks_per_w % 2 == 0
    mesh = plsc.VectorSubcoreMesh(core_axis_name="c", subcore_axis_name="s")

    @functools.partial(
        pl.kernel,
        mesh=mesh,
        out_type=jax.ShapeDtypeStruct((bc, K_DIM), jnp.float32),
        scratch_types=[
            pltpu.VMEM((blocks_per_w * IPB,), jnp.int32),
            pltpu.VMEM((IPB, EMB_SIZE), jnp.float32),
            pltpu.VMEM((IPB, EMB_SIZE), jnp.float32),
            pltpu.VMEM_SHARED((ACTIONS_MAX, EMB_SIZE), jnp.float32),
            pltpu.SemaphoreType.DMA,
            pltpu.SemaphoreType.DMA,
            pltpu.SemaphoreType.DMA,
            pltpu.SemaphoreType.DMA,
        ],
    )
    def k(idx_hbm, tab_hbm, out_hbm, idx_all, rows0, rows1, tab_sh,
          gsem0, gsem1, wsem0, wsem1):
        rows = (rows0, rows1)
        gsem = (gsem0, gsem1)
        wsem = (wsem0, wsem1)
        sid = lax.axis_index("s")
        wid = sid * nc + lax.axis_index("c")
        blk0 = wid * blocks_per_w

        @pl.when(sid == 0)
        def _():
            pltpu.sync_copy(tab_hbm, tab_sh)

        pltpu.sync_copy(idx_hbm.at[pl.ds(blk0 * IPB, blocks_per_w * IPB)],
                        idx_all)
        plsc.subcore_barrier()

        def fire_gather(k_, b):
            for h in range(2):
                pltpu.async_copy(
                    tab_sh.at[idx_all.at[pl.ds(k_ * IPB + h * HALF, HALF)]],
                    rows[b].at[pl.ds(h * HALF, HALF)], gsem[b])

        def wait_gather(k_, b):
            for h in range(2):
                pltpu.make_async_copy(
                    tab_sh.at[idx_all.at[pl.ds(k_ * IPB + h * HALF, HALF)]],
                    rows[b].at[pl.ds(h * HALF, HALF)], gsem[b]).wait()

        def fire_wb(k_, b):
            pltpu.async_copy(
                rows[b].reshape(NB, K_DIM),
                out_hbm.at[pl.ds((blk0 + k_) * NB, NB)], wsem[b])

        def wait_wb(k_, b):
            pltpu.make_async_copy(
                rows[b].reshape(NB, K_DIM),
                out_hbm.at[pl.ds((blk0 + k_) * NB, NB)], wsem[b]).wait()

        fire_gather(0, 0)
        fire_gather(1, 1)
        wait_gather(0, 0)
        fire_wb(0, 0)

        @pl.loop(2, blocks_per_w, step=2)
        def _(i):
            for b in range(2):
                k_ = i + b
                wait_wb(k_ - 2, b)
                fire_gather(k_, b)
                wait_gather(k_ - 1, 1 - b)
                fire_wb(k_ - 1, 1 - b)

        wait_gather(blocks_per_w - 1, 1)
        fire_wb(blocks_per_w - 1, 1)
        wait_wb(blocks_per_w - 2, 0)
        wait_wb(blocks_per_w - 1, 1)

    return k(idx, tab)


def _tc_matmul_chunk(x, w, b2, buf, chunk):
    def mm(x_ref, w_ref, b_ref, *rest):
        o_ref = rest[-1]
        acc = jnp.dot(x_ref[...].astype(jnp.bfloat16), w_ref[...],
                      preferred_element_type=jnp.float32)
        o_ref[...] = acc + b_ref[...]

    blk0 = chunk * (BC // BM)
    in_specs = [
        pl.BlockSpec((BM, K_DIM), lambda i: (i, 0)),
        pl.BlockSpec((K_DIM, MLP_OUT), lambda i: (0, 0)),
        pl.BlockSpec((1, MLP_OUT), lambda i: (0, 0)),
    ]
    args = [x, w, b2]
    aliases = {}
    if buf is not None:
        in_specs.append(pl.BlockSpec(memory_space=pl.ANY))
        args.append(buf)
        aliases = {3: 0}
    return pl.pallas_call(
        mm,
        grid=(BC // BM,),
        in_specs=in_specs,
        out_specs=pl.BlockSpec((BM, MLP_OUT), lambda i: (blk0 + i, 0)),
        out_shape=jax.ShapeDtypeStruct((BATCH, MLP_OUT), jnp.float32),
        input_output_aliases=aliases,
    )(*args)


def kernel(discrete_actions, emb_table, W, b):
    idx = discrete_actions.reshape(-1).astype(jnp.int32)
    w_bf = W.astype(jnp.bfloat16)
    b2 = b.reshape(1, MLP_OUT)
    gs = []
    for c in range(NUM_CHUNKS):
        idx_c = lax.dynamic_slice_in_dim(idx, c * BC * NUM_AGENTS,
                                         BC * NUM_AGENTS)
        gs.append(_sc_gather(idx_c, emb_table, BC))
    buf = _tc_matmul_chunk(gs[0], w_bf, b2, None, 0)
    for c in range(1, NUM_CHUNKS):
        buf = _tc_matmul_chunk(gs[c], w_bf, b2, buf, c)
    return buf


# speedup vs baseline: 1.1012x; 1.1012x over previous
import functools

import jax
import jax.numpy as jnp
from jax import lax
from jax.experimental import pallas as pl
from jax.experimental.pallas import tpu as pltpu
from jax.experimental.pallas import tpu_sc as plsc

ACTIONS_MAX = 1000
EMB_SIZE = 128
NUM_AGENTS = 26
MLP_OUT = 1024
BATCH = 16384
K_DIM = NUM_AGENTS * EMB_SIZE

NB = 8
IPB = NB * NUM_AGENTS
HALF = IPB // 2

BM = 1024
NUM_CHUNKS = 4
BC = BATCH // NUM_CHUNKS


def _sc_gather(idx, tab, bc):
    nblocks = bc // NB
    info = plsc.get_sparse_core_info()
    nc, ns = info.num_cores, info.num_subcores
    nw = nc * ns
    blocks_per_w = nblocks // nw
    assert nblocks % nw == 0 and blocks_per_w % 2 == 0
    mesh = plsc.VectorSubcoreMesh(core_axis_name="c", subcore_axis_name="s")

    @functools.partial(
        pl.kernel,
        mesh=mesh,
        out_type=jax.ShapeDtypeStruct((bc, K_DIM), jnp.float32),
        scratch_types=[
            pltpu.VMEM((blocks_per_w * IPB,), jnp.int32),
            pltpu.VMEM((IPB, EMB_SIZE), jnp.float32),
            pltpu.VMEM((IPB, EMB_SIZE), jnp.float32),
            pltpu.VMEM_SHARED((ACTIONS_MAX, EMB_SIZE), jnp.float32),
            pltpu.SemaphoreType.DMA,
            pltpu.SemaphoreType.DMA,
            pltpu.SemaphoreType.DMA,
            pltpu.SemaphoreType.DMA,
        ],
    )
    def k(idx_hbm, tab_hbm, out_hbm, idx_all, rows0, rows1, tab_sh,
          gsem0, gsem1, wsem0, wsem1):
        rows = (rows0, rows1)
        gsem = (gsem0, gsem1)
        wsem = (wsem0, wsem1)
        sid = lax.axis_index("s")
        wid = sid * nc + lax.axis_index("c")
        blk0 = wid * blocks_per_w

        @pl.when(sid == 0)
        def _():
            pltpu.sync_copy(tab_hbm, tab_sh)

        pltpu.sync_copy(idx_hbm.at[pl.ds(blk0 * IPB, blocks_per_w * IPB)],
                        idx_all)
        plsc.subcore_barrier()

        def fire_gather(k_, b):
            for h in range(2):
                pltpu.async_copy(
                    tab_sh.at[idx_all.at[pl.ds(k_ * IPB + h * HALF, HALF)]],
                    rows[b].at[pl.ds(h * HALF, HALF)], gsem[b])

        def wait_gather(k_, b):
            for h in range(2):
                pltpu.make_async_copy(
                    tab_sh.at[idx_all.at[pl.ds(k_ * IPB + h * HALF, HALF)]],
                    rows[b].at[pl.ds(h * HALF, HALF)], gsem[b]).wait()

        def fire_wb(k_, b):
            pltpu.async_copy(
                rows[b].reshape(NB, K_DIM),
                out_hbm.at[pl.ds((blk0 + k_) * NB, NB)], wsem[b])

        def wait_wb(k_, b):
            pltpu.make_async_copy(
                rows[b].reshape(NB, K_DIM),
                out_hbm.at[pl.ds((blk0 + k_) * NB, NB)], wsem[b]).wait()

        fire_gather(0, 0)
        fire_gather(1, 1)
        wait_gather(0, 0)
        fire_wb(0, 0)

        @pl.loop(2, blocks_per_w, step=2)
        def _(i):
            for b in range(2):
                k_ = i + b
                wait_wb(k_ - 2, b)
                fire_gather(k_, b)
                wait_gather(k_ - 1, 1 - b)
                fire_wb(k_ - 1, 1 - b)

        wait_gather(blocks_per_w - 1, 1)
        fire_wb(blocks_per_w - 1, 1)
        wait_wb(blocks_per_w - 2, 0)
        wait_wb(blocks_per_w - 1, 1)

    return k(idx, tab)


def _tc_matmul_chunk(x, w, b2, buf, chunk):
    def mm(x_ref, w_ref, b_ref, *rest):
        o_ref = rest[-1]
        acc = jnp.dot(x_ref[...].astype(jnp.bfloat16), w_ref[...],
                      preferred_element_type=jnp.float32)
        o_ref[...] = acc + b_ref[...]

    blk0 = chunk * (BC // BM)
    in_specs = [
        pl.BlockSpec((BM, K_DIM), lambda i: (i, 0)),
        pl.BlockSpec((K_DIM, MLP_OUT), lambda i: (0, 0)),
        pl.BlockSpec((1, MLP_OUT), lambda i: (0, 0)),
    ]
    args = [x, w, b2]
    aliases = {}
    if buf is not None:
        in_specs.append(pl.BlockSpec(memory_space=pl.ANY))
        args.append(buf)
        aliases = {3: 0}
    return pl.pallas_call(
        mm,
        grid=(BC // BM,),
        in_specs=in_specs,
        out_specs=pl.BlockSpec((BM, MLP_OUT), lambda i: (blk0 + i, 0)),
        out_shape=jax.ShapeDtypeStruct((BATCH, MLP_OUT), jnp.float32),
        input_output_aliases=aliases,
    )(*args)


def kernel(discrete_actions, emb_table, W, b):
    idx = discrete_actions.reshape(-1).astype(jnp.int32)
    w_bf = W.astype(jnp.bfloat16)
    b2 = b.reshape(1, MLP_OUT)
    gs = []
    for c in range(NUM_CHUNKS):
        idx_c = lax.dynamic_slice_in_dim(idx, c * BC * NUM_AGENTS,
                                         BC * NUM_AGENTS)
        gs.append(_sc_gather(idx_c, emb_table, BC))
    buf = _tc_matmul_chunk(gs[0], w_bf, b2, None, 0)
    for c in range(1, NUM_CHUNKS):
        buf = _tc_matmul_chunk(gs[c], w_bf, b2, buf, c)
    return buf
